# Initial kernel scaffold; baseline (speedup 1.0000x reference)
import jax
import jax.numpy as jnp
from jax.experimental import pallas as pl


def kernel(xyz, params):
    raise NotImplementedError("write your pallas kernel here")


# speedup vs baseline: 8.0582x; 8.0582x over previous
import jax
import jax.numpy as jnp
import numpy as np
from jax.experimental import pallas as pl

F32 = jnp.float32
_INTERPRET = False
_EXACT = jax.lax.Precision.HIGHEST


def _gdot(oh, table):
    return jnp.dot(oh, table, preferred_element_type=F32, precision=_EXACT)


def _xla_sum_k(x3):
    k = x3.shape[0]
    if k == 10:
        pairs = [x3[j] + x3[j + 1] for j in range(0, 10, 2)]
        acc = pairs[0]
        for t in pairs[1:]:
            acc = acc + t
        return acc
    w = []
    for r in range(8):
        acc = x3[r]
        for base in range(r + 8, k, 8):
            acc = acc + x3[base]
        w.append(acc)
    s = 4
    while s >= 1:
        for j in range(s):
            w[j] = w[j] + w[j + s]
        s //= 2
    return w[0]


def _pcall(body, grid, in_specs, out_specs, out_shape):
    return pl.pallas_call(
        body, grid=grid, in_specs=in_specs, out_specs=out_specs,
        out_shape=out_shape, interpret=_INTERPRET)


def _knn(qxyz, kxyzT, k, TM):
    B, M, _ = qxyz.shape
    N = kxyzT.shape[2]

    def body(qref, kref, oref):
        q = qref[0]
        kT = kref[0]
        d = None
        for c in range(3):
            diff = q[:, c:c + 1] - kT[c:c + 1, :]
            sq = diff * diff
            d = sq if d is None else d + sq
        iota = jax.lax.broadcasted_iota(jnp.int32, (TM, N), 1)
        cols = []
        for j in range(16):
            if j < k:
                m = jnp.min(d, axis=1, keepdims=True)
                jidx = jnp.min(jnp.where(d <= m, iota, N), axis=1, keepdims=True)
                d = jnp.where(iota == jidx, jnp.float32(np.inf), d)
                cols.append(jidx)
            else:
                cols.append(jnp.zeros((TM, 1), jnp.int32))
        oref[0] = jnp.concatenate(cols, axis=1)

    return _pcall(
        body, (B, M // TM),
        [pl.BlockSpec((1, TM, 3), lambda b, i: (b, i, 0)),
         pl.BlockSpec((1, 3, N), lambda b, i: (b, 0, 0))],
        pl.BlockSpec((1, TM, 16), lambda b, i: (b, i, 0)),
        jax.ShapeDtypeStruct((B, M, 16), jnp.int32))(qxyz, kxyzT)


def _mlp2_rows(x, w1, b1, w2, b2):
    h = jax.nn.relu(jnp.dot(x, w1, preferred_element_type=F32) + b1)
    return jnp.dot(h, w2, preferred_element_type=F32) + b2


def _softmax_attend(a3, val3):
    amax = jnp.max(a3, axis=0, keepdims=True)
    e = jnp.exp(a3 - amax)
    s = _xla_sum_k(e)
    attn = e / s[None]
    return _xla_sum_k(attn * val3)


def _tb_pos(xyz, idx, W, k, TM):
    B, N, _ = xyz.shape
    dw1, db1, dw2, db2, gw1, gb1, gw2, gb2 = W
    d = dw1.shape[1]

    def body(xt_r, xf_r, idx_r, dw1r, db1r, dw2r, db2r, gw1r, gb1r, gw2r, gb2r, o_r):
        xt = xt_r[0]
        xf = xf_r[0]
        idx_t = idx_r[0]
        iota = jax.lax.broadcasted_iota(jnp.int32, (TM, N), 1)
        dx_l = []
        for j in range(k):
            oh = (iota == idx_t[:, j:j + 1]).astype(F32)
            nx = _gdot(oh, xf)
            dx_l.append(xt - nx)
        dxyz = jnp.concatenate(dx_l, axis=0)
        pe = _mlp2_rows(dxyz, dw1r[...], db1r[...], dw2r[...], db2r[...])
        a = _mlp2_rows(pe, gw1r[...], gb1r[...], gw2r[...], gb2r[...])
        res = _softmax_attend(a.reshape(k, TM, d), pe.reshape(k, TM, d))
        o_r[0] = res

    wspec = lambda arr: pl.BlockSpec(arr.shape, lambda b, i: (0,) * arr.ndim)
    return _pcall(
        body, (B, N // TM),
        [pl.BlockSpec((1, TM, 3), lambda b, i: (b, i, 0)),
         pl.BlockSpec((1, N, 3), lambda b, i: (b, 0, 0)),
         pl.BlockSpec((1, TM, 16), lambda b, i: (b, i, 0))]
        + [wspec(w) for w in W],
        pl.BlockSpec((1, TM, d), lambda b, i: (b, i, 0)),
        jax.ShapeDtypeStruct((B, N, d), F32))(xyz, xyz, idx, *W)


def _tb_full(xyz, idx, q, kf, vf, feats, W, k, TM):
    B, N, d = q.shape
    dw1, db1, dw2, db2, gw1, gb1, gw2, gb2 = W

    def body(xt_r, xf_r, idx_r, q_r, kf_r, vf_r, ft_r,
             dw1r, db1r, dw2r, db2r, gw1r, gb1r, gw2r, gb2r, o_r):
        xt = xt_r[0]
        xf = xf_r[0]
        idx_t = idx_r[0]
        qv = q_r[0]
        kf_f = kf_r[0]
        vf_f = vf_r[0]
        ft = ft_r[0]
        iota = jax.lax.broadcasted_iota(jnp.int32, (TM, N), 1)
        dx_l, kk_l, vv_l = [], [], []
        for j in range(k):
            oh = (iota == idx_t[:, j:j + 1]).astype(F32)
            nx = _gdot(oh, xf)
            dx_l.append(xt - nx)
            kk_l.append(_gdot(oh, kf_f))
            vv_l.append(_gdot(oh, vf_f))
        dxyz = jnp.concatenate(dx_l, axis=0)
        kk = jnp.concatenate(kk_l, axis=0)
        vv = jnp.concatenate(vv_l, axis=0)
        pe = _mlp2_rows(dxyz, dw1r[...], db1r[...], dw2r[...], db2r[...])
        qrep = jnp.concatenate([qv] * k, axis=0)
        a = _mlp2_rows(qrep - kk + pe, gw1r[...], gb1r[...], gw2r[...], gb2r[...])
        res = _softmax_attend(a.reshape(k, TM, d),
                              vv.reshape(k, TM, d) + pe.reshape(k, TM, d))
        o_r[0] = res + ft

    wspec = lambda arr: pl.BlockSpec(arr.shape, lambda b, i: (0,) * arr.ndim)
    return _pcall(
        body, (B, N // TM),
        [pl.BlockSpec((1, TM, 3), lambda b, i: (b, i, 0)),
         pl.BlockSpec((1, N, 3), lambda b, i: (b, 0, 0)),
         pl.BlockSpec((1, TM, 16), lambda b, i: (b, i, 0)),
         pl.BlockSpec((1, TM, d), lambda b, i: (b, i, 0)),
         pl.BlockSpec((1, N, d), lambda b, i: (b, 0, 0)),
         pl.BlockSpec((1, N, d), lambda b, i: (b, 0, 0)),
         pl.BlockSpec((1, TM, d), lambda b, i: (b, i, 0))]
        + [wspec(w) for w in W],
        pl.BlockSpec((1, TM, d), lambda b, i: (b, i, 0)),
        jax.ShapeDtypeStruct((B, N, d), F32))(xyz, xyz, idx, q, kf, vf, feats, *W)


def _tb_all(xyz, q, kf, vf, feats, W, TM):
    B, N, d = q.shape
    dw1, db1, dw2, db2, gw1, gb1, gw2, gb2 = W

    def body(xt_r, xf_r, q_r, kf_r, vf_r, ft_r,
             dw1r, db1r, dw2r, db2r, gw1r, gb1r, gw2r, gb2r, o_r):
        xt = xt_r[0]
        xf = xf_r[0]
        qv = q_r[0]
        kf_f = kf_r[0]
        vf_f = vf_r[0]
        ft = ft_r[0]
        dxyz3 = xf[:, None, :] - xt[None, :, :]
        pe = _mlp2_rows(dxyz3.reshape(N * TM, 3) * -1.0,
                        dw1r[...], db1r[...], dw2r[...], db2r[...])
        kk3 = jnp.broadcast_to(kf_f[:, None, :], (N, TM, d))
        t = qv[None, :, :] - kk3 + pe.reshape(N, TM, d)
        a = _mlp2_rows(t.reshape(N * TM, d), gw1r[...], gb1r[...], gw2r[...], gb2r[...])
        vv3 = jnp.broadcast_to(vf_f[:, None, :], (N, TM, d))
        res = _softmax_attend(a.reshape(N, TM, d), vv3 + pe.reshape(N, TM, d))
        o_r[0] = res + ft

    wspec = lambda arr: pl.BlockSpec(arr.shape, lambda b, i: (0,) * arr.ndim)
    return _pcall(
        body, (B, N // TM),
        [pl.BlockSpec((1, TM, 3), lambda b, i: (b, i, 0)),
         pl.BlockSpec((1, N, 3), lambda b, i: (b, 0, 0)),
         pl.BlockSpec((1, TM, d), lambda b, i: (b, i, 0)),
         pl.BlockSpec((1, N, d), lambda b, i: (b, 0, 0)),
         pl.BlockSpec((1, N, d), lambda b, i: (b, 0, 0)),
         pl.BlockSpec((1, TM, d), lambda b, i: (b, i, 0))]
        + [wspec(w) for w in W],
        pl.BlockSpec((1, TM, d), lambda b, i: (b, i, 0)),
        jax.ShapeDtypeStruct((B, N, d), F32))(xyz, xyz, q, kf, vf, feats, *W)


def _lin2(x2, w, b):
    R, din = x2.shape
    dout = w.shape[1]
    if b is None:
        def body(x_r, w_r, o_r):
            o_r[...] = jnp.dot(x_r[...], w_r[...], preferred_element_type=F32)
        args, specs = (x2, w), [pl.BlockSpec(x2.shape, lambda: (0, 0)),
                                pl.BlockSpec(w.shape, lambda: (0, 0))]
    else:
        def body(x_r, w_r, b_r, o_r):
            o_r[...] = jnp.dot(x_r[...], w_r[...], preferred_element_type=F32) + b_r[...]
        args, specs = (x2, w, b), [pl.BlockSpec(x2.shape, lambda: (0, 0)),
                                   pl.BlockSpec(w.shape, lambda: (0, 0)),
                                   pl.BlockSpec(b.shape, lambda: (0, 0))]
    return _pcall(body, (), specs, pl.BlockSpec((R, dout), lambda: (0, 0)),
                  jax.ShapeDtypeStruct((R, dout), F32))(*args)


def _bn2(x2, m, v, g, b, relu=False, res=None):
    R, d = x2.shape

    def norm(x, m_v, v_v, g_v, b_v):
        y = (x - m_v) / jnp.sqrt(v_v + 1e-5) * g_v + b_v
        return jax.nn.relu(y) if relu else y

    s2 = lambda a: pl.BlockSpec(a.shape, lambda: (0, 0))
    if res is None:
        def body(x_r, m_r, v_r, g_r, b_r, o_r):
            o_r[...] = norm(x_r[...], m_r[...], v_r[...], g_r[...], b_r[...])
        args = (x2, m, v, g, b)
    else:
        def body(x_r, m_r, v_r, g_r, b_r, r_r, o_r):
            o_r[...] = r_r[...] + norm(x_r[...], m_r[...], v_r[...], g_r[...], b_r[...])
        args = (x2, m, v, g, b, res)
    return _pcall(body, (), [s2(a) for a in args],
                  pl.BlockSpec((R, d), lambda: (0, 0)),
                  jax.ShapeDtypeStruct((R, d), F32))(*args)


def _gather_max(idx, pts, ori, k, TM):
    B, M, _ = idx.shape
    N, d = pts.shape[1], pts.shape[2]

    def body(idx_r, pts_r, ori_r, o_r):
        idx_t = idx_r[0]
        pts_f = pts_r[0]
        iota = jax.lax.broadcasted_iota(jnp.int32, (TM, N), 1)
        acc = jnp.full((TM, d), -jnp.inf, F32)
        for j in range(k):
            oh = (iota == idx_t[:, j:j + 1]).astype(F32)
            acc = jnp.maximum(acc, _gdot(oh, pts_f))
        o_r[0] = acc + ori_r[0]

    return _pcall(
        body, (B, M // TM),
        [pl.BlockSpec((1, TM, 16), lambda b, i: (b, i, 0)),
         pl.BlockSpec((1, N, d), lambda b, i: (b, 0, 0)),
         pl.BlockSpec((1, TM, d), lambda b, i: (b, i, 0))],
        pl.BlockSpec((1, TM, d), lambda b, i: (b, i, 0)),
        jax.ShapeDtypeStruct((B, M, d), F32))(idx, pts, ori)


def _head(f2, w1, b1, w2, b2, g, b):
    B, N, d = f2.shape

    def body(x_r, w1r, b1r, w2r, b2r, g_r, b_r, o_r):
        glob = jnp.max(x_r[...], axis=1)
        lat = _mlp2_rows(glob, w1r[...], b1r[...], w2r[...], b2r[...])
        m = jnp.mean(lat, axis=0, keepdims=True)
        v = jnp.mean((lat - m) ** 2, axis=0, keepdims=True)
        o_r[...] = (lat - m) / jnp.sqrt(v + 1e-5) * g_r[...] + b_r[...]

    specs = [pl.BlockSpec(f2.shape, lambda: (0, 0, 0))] + \
            [pl.BlockSpec(a.shape, lambda: (0, 0)) for a in (w1, b1, w2, b2, g, b)]
    return _pcall(body, (), specs, pl.BlockSpec((B, d), lambda: (0, 0)),
                  jax.ShapeDtypeStruct((B, d), F32))(f2, w1, b1, w2, b2, g, b)


def _r2(a):
    return a.reshape(1, -1)


def _tb_weights(p, pre):
    return (p[pre + '_dw1'], _r2(p[pre + '_db1']), p[pre + '_dw2'], _r2(p[pre + '_db2']),
            p[pre + '_gw1'], _r2(p[pre + '_gb1']), p[pre + '_gw2'], _r2(p[pre + '_gb2']))


def _bn3(x3, g, b, relu=False, res=None):
    B, N, d = x3.shape
    m = jnp.mean(x3, axis=(0, 1)).reshape(1, d)
    v = jnp.var(x3, axis=(0, 1)).reshape(1, d)
    r2 = None if res is None else res.reshape(B * N, d)
    y = _bn2(x3.reshape(B * N, d), m, v, _r2(g), _r2(b), relu=relu, res=r2)
    return y.reshape(B, N, d)


def _lin3(x3, w, b):
    B, N, din = x3.shape
    y = _lin2(x3.reshape(B * N, din), w, None if b is None else _r2(b))
    return y.reshape(B, N, w.shape[1])


def _qkv(feats, p, pre):
    d = feats.shape[-1]
    wqkv = jnp.concatenate([p[pre + '_wq'], p[pre + '_wk'], p[pre + '_wv']], axis=1)
    qkv = _lin3(feats, wqkv, None)
    return qkv[..., :d], qkv[..., d:2 * d], qkv[..., 2 * d:]


def _td_stage(xyz, feats, p, pre, npoint, nneigh, TM_knn, TM_g):
    B, N, _ = xyz.shape
    stride = N // npoint
    new_xyz = xyz[:, ::stride, :]
    pts = _lin3(feats, p[pre + '_f1w'], p[pre + '_f1b'])
    pts_ori = pts[:, ::stride, :]
    h = _bn3(_lin3(pts, p[pre + '_c1w'], p[pre + '_c1b']),
             p[pre + '_bn1g'], p[pre + '_bn1b'], relu=True)
    pts2 = _bn3(_lin3(h, p[pre + '_c2w'], p[pre + '_c2b']),
                p[pre + '_bn2g'], p[pre + '_bn2b'], relu=True, res=pts)
    idx = _knn(new_xyz, jnp.swapaxes(xyz, 1, 2), nneigh, TM_knn)
    new_pts = _gather_max(idx, pts2, pts_ori, nneigh, TM_g)
    return new_xyz, _bn3(new_pts, p[pre + '_bng'], p[pre + '_bnb'])


def _tb_stage(xyz, feats, p, pre, k, TM):
    idx = _knn(xyz, jnp.swapaxes(xyz, 1, 2), k, TM)
    q, kf, vf = _qkv(feats, p, pre)
    res = _tb_full(xyz, idx, q, kf, vf, feats, _tb_weights(p, pre), k, TM)
    return _bn3(res, p[pre + '_bng'], p[pre + '_bnb'])


def kernel(xyz, params):
    p = params
    idx0 = _knn(xyz, jnp.swapaxes(xyz, 1, 2), 10, 256)
    res0 = _tb_pos(xyz, idx0, _tb_weights(p, 'tb0'), 10, 256)
    f0 = _bn3(res0, p['tb0_bng'], p['tb0_bnb'])
    xyz1, f1 = _td_stage(xyz, f0, p, 'td1', 512, 16, 256, 256)
    f1 = _tb_stage(xyz1, f1, p, 'tb1', 16, 256)
    xyz2, f2 = _td_stage(xyz1, f1, p, 'td2', 128, 16, 128, 128)
    f2 = _tb_stage(xyz2, f2, p, 'tb2', 16, 128)
    for pre in ('tf0', 'tf1'):
        q, kf, vf = _qkv(f2, p, pre)
        res = _tb_all(xyz2, q, kf, vf, f2, _tb_weights(p, pre), 32)
        f2 = _bn3(res, p[pre + '_bng'], p[pre + '_bnb'])
    return _head(f2, p['fm_w1'], _r2(p['fm_b1']), p['fm_w2'], _r2(p['fm_b2']),
                 _r2(p['fm_bng']), _r2(p['fm_bnb']))
